# baseline (speedup 1.0000x reference)
import functools

import jax
import jax.numpy as jnp
from jax import lax
from jax.experimental import pallas as pl
from jax.experimental.pallas import tpu as pltpu
from jax.experimental.pallas import tpu_sc as plsc

_CHUNK = 128
_TBLK = 4096


def _tc_transpose_pack2(Ut8):
    A, D8, N = Ut8.shape
    D = A * D8
    H = _TBLK // 2
    NB = -(-N // _TBLK)

    def body(*refs):
        out_ref = refs[-1]
        y = jnp.concatenate(
            [jnp.transpose(refs[a][0], (1, 0)) for a in range(A)], axis=1
        )
        out_ref[...] = jnp.concatenate([y[:H], y[H:]], axis=1)

    def mk_spec(a):
        return pl.BlockSpec((1, D8, _TBLK), lambda i, a=a: (a, 0, i))

    return pl.pallas_call(
        body,
        grid=(NB,),
        in_specs=[mk_spec(a) for a in range(A)],
        out_specs=pl.BlockSpec((H, 2 * D), lambda i: (i, 0)),
        out_shape=jax.ShapeDtypeStruct((NB * H, 2 * D), jnp.float32),
    )(*([Ut8] * A))


def _sc_gather(table, idx, *, width):
    NW, n_chunks, _ = idx.shape
    b_per_w = n_chunks * _CHUNK
    B = NW * b_per_w
    info = plsc.get_sparse_core_info()
    NC = info.num_cores

    mesh = plsc.VectorSubcoreMesh(core_axis_name="c", subcore_axis_name="s")

    @functools.partial(
        pl.kernel,
        mesh=mesh,
        out_type=jax.ShapeDtypeStruct((B, width), jnp.float32),
        scratch_types=[
            pltpu.VMEM((n_chunks, _CHUNK), jnp.int32),
            pltpu.VMEM((b_per_w, width), jnp.float32),
            pltpu.SemaphoreType.DMA,
        ],
        compiler_params=pltpu.CompilerParams(use_tc_tiling_on_sc=False),
    )
    def k(idx_hbm, tab_hbm, out_hbm, idx_v, rows_v, sem):
        wid = lax.axis_index("s") * NC + lax.axis_index("c")
        base = wid * b_per_w
        pltpu.sync_copy(idx_hbm.at[wid], idx_v)
        copies = []
        for c in range(n_chunks):
            copies.append(
                pltpu.async_copy(
                    tab_hbm.at[idx_v.at[c]],
                    rows_v.at[pl.ds(c * _CHUNK, _CHUNK)],
                    sem,
                )
            )
        for cp in copies:
            cp.wait()
        pltpu.sync_copy(rows_v, out_hbm.at[pl.ds(base, b_per_w)])

    return k(idx, table)


def kernel(ij, U, V):
    B = ij.shape[0]
    dim = U.shape[1]
    info = plsc.get_sparse_core_info()
    NW = info.num_cores * info.num_subcores
    b_per_w = B // NW
    n_chunks = b_per_w // _CHUNK
    idx = ij.astype(jnp.int32)
    iu = idx[:, 0]
    iv = idx[:, 1].reshape(NW, n_chunks, _CHUNK)

    H = _TBLK // 2
    r = iu % _TBLK
    half = r // H
    ipair = ((iu // _TBLK) * H + r % H).reshape(NW, n_chunks, _CHUNK)

    Ut8 = jnp.swapaxes(U, 0, 1).reshape(8, dim // 8, U.shape[0])
    Up = _tc_transpose_pack2(Ut8)
    pair_rows = _sc_gather(Up, ipair, width=2 * dim)
    u_rows = jnp.where((half == 1)[:, None], pair_rows[:, dim:], pair_rows[:, :dim])
    v_rows = _sc_gather(V, iv, width=dim)
    return (u_rows, v_rows)


# speedup vs baseline: 2.3598x; 2.3598x over previous
import functools

import jax
import jax.numpy as jnp
from jax import lax
from jax.experimental import pallas as pl
from jax.experimental.pallas import tpu as pltpu
from jax.experimental.pallas import tpu_sc as plsc

_CHUNK = 128


def _sc_gather(table, idx, *, width):
    NW, n_chunks, _ = idx.shape
    b_per_w = n_chunks * _CHUNK
    B = NW * b_per_w
    info = plsc.get_sparse_core_info()
    NC = info.num_cores

    mesh = plsc.VectorSubcoreMesh(core_axis_name="c", subcore_axis_name="s")

    @functools.partial(
        pl.kernel,
        mesh=mesh,
        out_type=jax.ShapeDtypeStruct((B, width), jnp.float32),
        scratch_types=[
            pltpu.VMEM((n_chunks, _CHUNK), jnp.int32),
            pltpu.VMEM((b_per_w, width), jnp.float32),
            pltpu.SemaphoreType.DMA,
        ],
        compiler_params=pltpu.CompilerParams(use_tc_tiling_on_sc=False),
    )
    def k(idx_hbm, tab_hbm, out_hbm, idx_v, rows_v, sem):
        wid = lax.axis_index("s") * NC + lax.axis_index("c")
        base = wid * b_per_w
        pltpu.sync_copy(idx_hbm.at[wid], idx_v)
        copies = []
        for c in range(n_chunks):
            copies.append(
                pltpu.async_copy(
                    tab_hbm.at[idx_v.at[c]],
                    rows_v.at[pl.ds(c * _CHUNK, _CHUNK)],
                    sem,
                )
            )
        for cp in copies:
            cp.wait()
        pltpu.sync_copy(rows_v, out_hbm.at[pl.ds(base, b_per_w)])

    return k(idx, table)


def kernel(ij, U, V):
    B = ij.shape[0]
    dim = U.shape[1]
    info = plsc.get_sparse_core_info()
    NW = info.num_cores * info.num_subcores
    b_per_w = B // NW
    n_chunks = b_per_w // _CHUNK
    idx = ij.astype(jnp.int32)
    iu = idx[:, 0].reshape(NW, n_chunks, _CHUNK)
    iv = idx[:, 1].reshape(NW, n_chunks, _CHUNK)
    u_rows = _sc_gather(U, iu, width=dim)
    v_rows = _sc_gather(V, iv, width=dim)
    return (u_rows, v_rows)


# speedup vs baseline: 3.4124x; 1.4461x over previous
import functools

import jax
import jax.numpy as jnp
from jax import lax
from jax.experimental import pallas as pl
from jax.experimental.pallas import tpu as pltpu
from jax.experimental.pallas import tpu_sc as plsc

_CHUNK = 128
_TBLK = 8192


def _tc_transpose_pack2(Ut):
    D, N = Ut.shape
    H = _TBLK // 2
    NB = -(-N // _TBLK)

    def body(ut_ref, out_ref):
        y = jnp.transpose(ut_ref[...], (1, 0))
        out_ref[...] = jnp.concatenate([y[:H], y[H:]], axis=1)

    return pl.pallas_call(
        body,
        grid=(NB,),
        in_specs=[pl.BlockSpec((D, _TBLK), lambda i: (0, i))],
        out_specs=pl.BlockSpec((H, 2 * D), lambda i: (i, 0)),
        out_shape=jax.ShapeDtypeStruct((NB * H, 2 * D), jnp.float32),
    )(Ut)


def _sc_gather(table, idx, *, width):
    NW, n_chunks, _ = idx.shape
    b_per_w = n_chunks * _CHUNK
    B = NW * b_per_w
    info = plsc.get_sparse_core_info()
    NC = info.num_cores

    mesh = plsc.VectorSubcoreMesh(core_axis_name="c", subcore_axis_name="s")

    @functools.partial(
        pl.kernel,
        mesh=mesh,
        out_type=jax.ShapeDtypeStruct((B, width), jnp.float32),
        scratch_types=[
            pltpu.VMEM((n_chunks, _CHUNK), jnp.int32),
            pltpu.VMEM((b_per_w, width), jnp.float32),
            pltpu.SemaphoreType.DMA,
        ],
        compiler_params=pltpu.CompilerParams(use_tc_tiling_on_sc=False),
    )
    def k(idx_hbm, tab_hbm, out_hbm, idx_v, rows_v, sem):
        wid = lax.axis_index("s") * NC + lax.axis_index("c")
        base = wid * b_per_w
        pltpu.sync_copy(idx_hbm.at[wid], idx_v)
        copies = []
        for c in range(n_chunks):
            copies.append(
                pltpu.async_copy(
                    tab_hbm.at[idx_v.at[c]],
                    rows_v.at[pl.ds(c * _CHUNK, _CHUNK)],
                    sem,
                )
            )
        for cp in copies:
            cp.wait()
        pltpu.sync_copy(rows_v, out_hbm.at[pl.ds(base, b_per_w)])

    return k(idx, table)


def kernel(ij, U, V):
    B = ij.shape[0]
    dim = U.shape[1]
    info = plsc.get_sparse_core_info()
    NW = info.num_cores * info.num_subcores
    b_per_w = B // NW
    n_chunks = b_per_w // _CHUNK
    idx = ij.astype(jnp.int32)
    iu = idx[:, 0]
    iv = idx[:, 1].reshape(NW, n_chunks, _CHUNK)

    H = _TBLK // 2
    r = iu % _TBLK
    half = r // H
    ipair = ((iu // _TBLK) * H + r % H).reshape(NW, n_chunks, _CHUNK)

    Up = _tc_transpose_pack2(jnp.swapaxes(U, 0, 1))
    pair_rows = _sc_gather(Up, ipair, width=2 * dim)
    u_rows = jnp.where((half == 1)[:, None], pair_rows[:, dim:], pair_rows[:, :dim])
    v_rows = _sc_gather(V, iv, width=dim)
    return (u_rows, v_rows)
